# Initial kernel scaffold; baseline (speedup 1.0000x reference)
import jax
import jax.numpy as jnp
from jax.experimental import pallas as pl


def kernel(x, edge_index, edge_attr, u, batch, W1, b1, gamma, beta, W2, b2):
    raise NotImplementedError("write your pallas kernel here")


# speedup vs baseline: 5.7433x; 5.7433x over previous
import functools

import jax
import jax.numpy as jnp
from jax import lax
from jax.experimental import pallas as pl
from jax.experimental.pallas import tpu as pltpu
from jax.experimental.pallas import tpu_sc as plsc

NUM_GRAPHS = 512
N_NODES = 100000
D_NODE = 128
D_U = 64
D_REP = 128

NC = 2
NS = 16
NW = NC * NS

CHUNK = 128
NCHUNK = (N_NODES + CHUNK - 1) // CHUNK
FULL_CHUNKS = N_NODES // CHUNK
TAIL = N_NODES - FULL_CHUNKS * CHUNK
BASE_CH = NCHUNK // NW
EXTRA = NCHUNK - BASE_CH * NW

CNT_BLK = 2048
CNT_STEPS = 49
N_PAD = CNT_BLK * CNT_STEPS
CNT_W = 8


def _sc_body(x_hbm, seg_hbm, zacc_hbm, sums_hbm, idx_v, rows_v, acc_sh):
    c = lax.axis_index("c")
    s = lax.axis_index("s")
    wid = s * NC + c

    @pl.when(s == 0)
    def _():
        pltpu.sync_copy(zacc_hbm, acc_sh)

    plsc.subcore_barrier()

    n_ch = BASE_CH + jnp.where(wid < EXTRA, 1, 0)

    def body(k, carry):
        chunk = k * NW + wid
        base = chunk * CHUNK
        pltpu.sync_copy(seg_hbm.at[pl.ds(base, CHUNK)], idx_v)

        @pl.when(chunk < FULL_CHUNKS)
        def _():
            pltpu.sync_copy(x_hbm.at[pl.ds(base, CHUNK)], rows_v)

        @pl.when(chunk == FULL_CHUNKS)
        def _():
            pltpu.sync_copy(x_hbm.at[pl.ds(FULL_CHUNKS * CHUNK, TAIL)],
                            rows_v.at[pl.ds(0, TAIL)])

        pltpu.sync_copy(rows_v, acc_sh.at[idx_v], add=True)
        return carry

    lax.fori_loop(0, n_ch, body, 0)

    plsc.subcore_barrier()

    @pl.when(s == 0)
    def _():
        pltpu.sync_copy(acc_sh.at[pl.ds(0, NUM_GRAPHS)], sums_hbm.at[c])


_sc_segment_sum = functools.partial(
    pl.kernel,
    out_type=jax.ShapeDtypeStruct((NC, NUM_GRAPHS, D_NODE), jnp.float32),
    mesh=plsc.VectorSubcoreMesh(core_axis_name="c", subcore_axis_name="s",
                                num_cores=NC, num_subcores=NS),
    scratch_types=(
        pltpu.VMEM((CHUNK,), jnp.int32),
        pltpu.VMEM((CHUNK, D_NODE), jnp.float32),
        pltpu.VMEM_SHARED((NUM_GRAPHS + 1, D_NODE), jnp.float32),
    ),
)(_sc_body)


def _cnt_body(ids_ref, o_ref):
    k = pl.program_id(0)
    ids = ids_ref[0]
    gids = lax.broadcasted_iota(jnp.int32, (NUM_GRAPHS, 1), 0)
    oh = (ids == gids).astype(jnp.float32)
    contrib = lax.dot(oh, jnp.ones((CNT_BLK, CNT_W), jnp.float32),
                      preferred_element_type=jnp.float32)

    @pl.when(k == 0)
    def _():
        o_ref[...] = jnp.zeros_like(o_ref)

    o_ref[...] += contrib


def _mlp_body(ps_ref, pc_ref, u_ref, W1_ref, b1_ref, g_ref, be_ref,
              W2_ref, b2_ref, o_ref):
    sums = ps_ref[0] + ps_ref[1]
    cnt = pc_ref[:, 0:1]
    mean = sums / jnp.maximum(cnt, 1.0)
    h = (lax.dot(u_ref[...], W1_ref[0:D_U, :],
                 precision=lax.Precision.HIGHEST,
                 preferred_element_type=jnp.float32)
         + lax.dot(mean, W1_ref[D_U:, :],
                   precision=lax.Precision.HIGHEST,
                   preferred_element_type=jnp.float32)
         + b1_ref[...])
    h = jnp.where(h >= 0, h, 0.01 * h)
    mu = jnp.mean(h, axis=-1, keepdims=True)
    var = jnp.mean((h - mu) ** 2, axis=-1, keepdims=True)
    h = (h - mu) * lax.rsqrt(var + 1e-5) * g_ref[...] + be_ref[...]
    o_ref[...] = (lax.dot(h, W2_ref[...],
                          precision=lax.Precision.HIGHEST,
                          preferred_element_type=jnp.float32)
                  + b2_ref[...])


def kernel(x, edge_index, edge_attr, u, batch, W1, b1, gamma, beta, W2, b2):
    del edge_index, edge_attr
    seg = batch.astype(jnp.int32)
    seg_pad = jnp.concatenate(
        [seg, jnp.full((N_PAD - N_NODES,), NUM_GRAPHS, jnp.int32)])
    zacc = jnp.zeros((NUM_GRAPHS + 1, D_NODE), jnp.float32)

    part_sums = _sc_segment_sum(x, seg_pad, zacc)

    counts = pl.pallas_call(
        _cnt_body,
        grid=(CNT_STEPS,),
        in_specs=[pl.BlockSpec((1, 1, CNT_BLK), lambda k: (k, 0, 0))],
        out_specs=pl.BlockSpec((NUM_GRAPHS, CNT_W), lambda k: (0, 0)),
        out_shape=jax.ShapeDtypeStruct((NUM_GRAPHS, CNT_W), jnp.float32),
    )(seg_pad.reshape(CNT_STEPS, 1, CNT_BLK))

    out = pl.pallas_call(
        _mlp_body,
        out_shape=jax.ShapeDtypeStruct((NUM_GRAPHS, D_REP), jnp.float32),
    )(part_sums, counts, u, W1,
      b1.reshape(1, D_REP), gamma.reshape(1, D_REP),
      beta.reshape(1, D_REP), W2, b2.reshape(1, D_REP))
    return out


# speedup vs baseline: 7.8620x; 1.3689x over previous
import functools

import jax
import jax.numpy as jnp
from jax import lax
from jax.experimental import pallas as pl
from jax.experimental.pallas import tpu as pltpu
from jax.experimental.pallas import tpu_sc as plsc

NUM_GRAPHS = 512
N_NODES = 100000
D_NODE = 128
D_U = 64
D_REP = 128

NC = 2
NS = 16
NW = NC * NS

ROWS_IT = 256
SUB = ROWS_IT // 128
NCHUNK = 392
N_PAD = NCHUNK * ROWS_IT
FULL_IT = N_NODES // ROWS_IT
TAIL_BASE = FULL_IT * ROWS_IT
TAIL_ROWS = N_NODES - TAIL_BASE
BASE_CH = FULL_IT // NW
EXTRA = FULL_IT - BASE_CH * NW
MAX_CH = BASE_CH + (1 if EXTRA else 0)
OUTER = (MAX_CH + 1) // 2

CNT_BLK = 2048
CNT_STEPS = N_PAD // CNT_BLK
CNT_W = 8


def _sc_body(x_hbm, seg_hbm, zacc_hbm, sums_hbm,
             idx_v, rows_v, acc_sh, sem_i0, sem_i1, sem_r0, sem_r1):
    c = lax.axis_index("c")
    s = lax.axis_index("s")
    wid = s * NC + c
    sem_i = (sem_i0, sem_i1)
    sem_r = (sem_r0, sem_r1)

    @pl.when(s == 0)
    def _():
        pltpu.sync_copy(zacc_hbm, acc_sh)

    plsc.subcore_barrier()

    n_ch = BASE_CH + jnp.where(wid < EXTRA, 1, 0)

    def start_in(ch, b):
        pltpu.async_copy(seg_hbm.at[ch], idx_v.at[b], sem_i[b])
        pltpu.async_copy(x_hbm.at[pl.ds(ch * ROWS_IT, ROWS_IT)],
                         rows_v.at[b], sem_r[b])

    def wait_in(ch, b):
        pltpu.make_async_copy(seg_hbm.at[ch], idx_v.at[b], sem_i[b]).wait()
        pltpu.make_async_copy(x_hbm.at[pl.ds(ch * ROWS_IT, ROWS_IT)],
                              rows_v.at[b], sem_r[b]).wait()

    def scatter(b):
        for j in range(SUB):
            pltpu.sync_copy(rows_v.at[b].at[pl.ds(j * 128, 128)],
                            acc_sh.at[idx_v.at[b, j]], add=True)

    start_in(wid, 0)

    def outer(g, carry):
        for b in range(2):
            k = g * 2 + b
            ch = k * NW + wid

            @pl.when(k < n_ch)
            def _():
                wait_in(ch, b)

                @pl.when(k + 1 < n_ch)
                def _():
                    start_in(ch + NW, 1 - b)

                scatter(b)
        return carry

    lax.fori_loop(0, OUTER, outer, 0)

    @pl.when(wid == 0)
    def _():
        pltpu.sync_copy(seg_hbm.at[FULL_IT], idx_v.at[0])
        pltpu.sync_copy(x_hbm.at[pl.ds(TAIL_BASE, TAIL_ROWS)],
                        rows_v.at[0].at[pl.ds(0, TAIL_ROWS)])
        scatter(0)

    plsc.subcore_barrier()

    @pl.when(s == 0)
    def _():
        pltpu.sync_copy(acc_sh.at[pl.ds(0, NUM_GRAPHS)], sums_hbm.at[c])


_sc_segment_sum = functools.partial(
    pl.kernel,
    out_type=jax.ShapeDtypeStruct((NC, NUM_GRAPHS, D_NODE), jnp.float32),
    mesh=plsc.VectorSubcoreMesh(core_axis_name="c", subcore_axis_name="s",
                                num_cores=NC, num_subcores=NS),
    scratch_types=(
        pltpu.VMEM((2, SUB, 128), jnp.int32),
        pltpu.VMEM((2, ROWS_IT, D_NODE), jnp.float32),
        pltpu.VMEM_SHARED((NUM_GRAPHS + 1, D_NODE), jnp.float32),
        pltpu.SemaphoreType.DMA,
        pltpu.SemaphoreType.DMA,
        pltpu.SemaphoreType.DMA,
        pltpu.SemaphoreType.DMA,
    ),
)(_sc_body)


def _cnt_body(ids_ref, o_ref):
    k = pl.program_id(0)
    ids = ids_ref[0]
    gids = lax.broadcasted_iota(jnp.int32, (NUM_GRAPHS, 1), 0)
    oh = (ids == gids).astype(jnp.float32)
    contrib = lax.dot(oh, jnp.ones((CNT_BLK, CNT_W), jnp.float32),
                      preferred_element_type=jnp.float32)

    @pl.when(k == 0)
    def _():
        o_ref[...] = jnp.zeros_like(o_ref)

    o_ref[...] += contrib


def _mlp_body(ps_ref, pc_ref, u_ref, W1_ref, b1_ref, g_ref, be_ref,
              W2_ref, b2_ref, o_ref):
    sums = ps_ref[0] + ps_ref[1]
    cnt = pc_ref[:, 0:1]
    mean = sums / jnp.maximum(cnt, 1.0)
    h = (lax.dot(u_ref[...], W1_ref[0:D_U, :],
                 precision=lax.Precision.HIGHEST,
                 preferred_element_type=jnp.float32)
         + lax.dot(mean, W1_ref[D_U:, :],
                   precision=lax.Precision.HIGHEST,
                   preferred_element_type=jnp.float32)
         + b1_ref[...])
    h = jnp.where(h >= 0, h, 0.01 * h)
    mu = jnp.mean(h, axis=-1, keepdims=True)
    var = jnp.mean((h - mu) ** 2, axis=-1, keepdims=True)
    h = (h - mu) * lax.rsqrt(var + 1e-5) * g_ref[...] + be_ref[...]
    o_ref[...] = (lax.dot(h, W2_ref[...],
                          precision=lax.Precision.HIGHEST,
                          preferred_element_type=jnp.float32)
                  + b2_ref[...])


def kernel(x, edge_index, edge_attr, u, batch, W1, b1, gamma, beta, W2, b2):
    del edge_index, edge_attr
    seg = batch.astype(jnp.int32)
    seg_pad = jnp.concatenate(
        [seg, jnp.full((N_PAD - N_NODES,), NUM_GRAPHS, jnp.int32)])
    zacc = jnp.zeros((NUM_GRAPHS + 1, D_NODE), jnp.float32)

    counts = pl.pallas_call(
        _cnt_body,
        grid=(CNT_STEPS,),
        in_specs=[pl.BlockSpec((1, 1, CNT_BLK), lambda k: (k, 0, 0))],
        out_specs=pl.BlockSpec((NUM_GRAPHS, CNT_W), lambda k: (0, 0)),
        out_shape=jax.ShapeDtypeStruct((NUM_GRAPHS, CNT_W), jnp.float32),
    )(seg_pad.reshape(CNT_STEPS, 1, CNT_BLK))

    part_sums = _sc_segment_sum(x, seg_pad.reshape(NCHUNK, SUB, 128), zacc)

    out = pl.pallas_call(
        _mlp_body,
        out_shape=jax.ShapeDtypeStruct((NUM_GRAPHS, D_REP), jnp.float32),
    )(part_sums, counts, u, W1,
      b1.reshape(1, D_REP), gamma.reshape(1, D_REP),
      beta.reshape(1, D_REP), W2, b2.reshape(1, D_REP))
    return out


# speedup vs baseline: 8.0719x; 1.0267x over previous
import functools

import jax
import jax.numpy as jnp
from jax import lax
from jax.experimental import pallas as pl
from jax.experimental.pallas import tpu as pltpu
from jax.experimental.pallas import tpu_sc as plsc

NUM_GRAPHS = 512
N_NODES = 100000
D_NODE = 128
D_U = 64
D_REP = 128

NC = 2
NS = 16
NW = NC * NS

ROWS_IT = 256
SUB = ROWS_IT // 128
CNT_BLK = 4096
CNT_STEPS = 25
N_PAD = CNT_BLK * CNT_STEPS
NCHUNK = N_PAD // ROWS_IT
FULL_IT = N_NODES // ROWS_IT
TAIL_BASE = FULL_IT * ROWS_IT
TAIL_ROWS = N_NODES - TAIL_BASE
BASE_CH = FULL_IT // NW
EXTRA = FULL_IT - BASE_CH * NW
MAX_CH = BASE_CH + (1 if EXTRA else 0)
OUTER = (MAX_CH + 1) // 2

CNT_HI = 32
CNT_LO = 16


def _sc_body(x_hbm, seg_hbm, zacc_hbm, sums_hbm,
             idx_v, rows_v, acc_sh, sem_i0, sem_i1, sem_r0, sem_r1):
    c = lax.axis_index("c")
    s = lax.axis_index("s")
    wid = s * NC + c
    sem_i = (sem_i0, sem_i1)
    sem_r = (sem_r0, sem_r1)

    @pl.when(s == 0)
    def _():
        pltpu.sync_copy(zacc_hbm, acc_sh)

    plsc.subcore_barrier()

    n_ch = BASE_CH + jnp.where(wid < EXTRA, 1, 0)

    def start_in(ch, b):
        pltpu.async_copy(seg_hbm.at[ch], idx_v.at[b], sem_i[b])
        pltpu.async_copy(x_hbm.at[pl.ds(ch * ROWS_IT, ROWS_IT)],
                         rows_v.at[b], sem_r[b])

    def wait_in(ch, b):
        pltpu.make_async_copy(seg_hbm.at[ch], idx_v.at[b], sem_i[b]).wait()
        pltpu.make_async_copy(x_hbm.at[pl.ds(ch * ROWS_IT, ROWS_IT)],
                              rows_v.at[b], sem_r[b]).wait()

    def scatter(b):
        for j in range(SUB):
            pltpu.sync_copy(rows_v.at[b].at[pl.ds(j * 128, 128)],
                            acc_sh.at[idx_v.at[b, j]], add=True)

    start_in(wid, 0)

    def outer(g, carry):
        for b in range(2):
            k = g * 2 + b
            ch = k * NW + wid

            @pl.when(k < n_ch)
            def _():
                wait_in(ch, b)

                @pl.when(k + 1 < n_ch)
                def _():
                    start_in(ch + NW, 1 - b)

                scatter(b)
        return carry

    lax.fori_loop(0, OUTER, outer, 0)

    @pl.when(wid == 0)
    def _():
        pltpu.sync_copy(seg_hbm.at[FULL_IT], idx_v.at[0])
        pltpu.sync_copy(x_hbm.at[pl.ds(TAIL_BASE, TAIL_ROWS)],
                        rows_v.at[0].at[pl.ds(0, TAIL_ROWS)])
        scatter(0)

    plsc.subcore_barrier()

    @pl.when(s == 0)
    def _():
        pltpu.sync_copy(acc_sh.at[pl.ds(0, NUM_GRAPHS)], sums_hbm.at[c])


_sc_segment_sum = functools.partial(
    pl.kernel,
    out_type=jax.ShapeDtypeStruct((NC, NUM_GRAPHS, D_NODE), jnp.float32),
    mesh=plsc.VectorSubcoreMesh(core_axis_name="c", subcore_axis_name="s",
                                num_cores=NC, num_subcores=NS),
    scratch_types=(
        pltpu.VMEM((2, SUB, 128), jnp.int32),
        pltpu.VMEM((2, ROWS_IT, D_NODE), jnp.float32),
        pltpu.VMEM_SHARED((NUM_GRAPHS + 1, D_NODE), jnp.float32),
        pltpu.SemaphoreType.DMA,
        pltpu.SemaphoreType.DMA,
        pltpu.SemaphoreType.DMA,
        pltpu.SemaphoreType.DMA,
    ),
)(_sc_body)


def _cnt_body(ids_ref, o_ref):
    k = pl.program_id(0)
    ids = ids_ref[0]
    hi = ids >> 4
    lo = ids & 15
    a = (hi == lax.broadcasted_iota(jnp.int32, (CNT_HI, 1), 0))
    b = (lo == lax.broadcasted_iota(jnp.int32, (CNT_LO, 1), 0))
    contrib = lax.dot_general(a.astype(jnp.float32), b.astype(jnp.float32),
                              (((1,), (1,)), ((), ())),
                              preferred_element_type=jnp.float32)

    @pl.when(k == 0)
    def _():
        o_ref[...] = jnp.zeros_like(o_ref)

    o_ref[...] += contrib


def _mlp_body(ps_ref, pc_ref, u_ref, W1_ref, b1_ref, g_ref, be_ref,
              W2_ref, b2_ref, o_ref):
    sums = ps_ref[0] + ps_ref[1]
    cnt = pc_ref[...]
    mean = sums / jnp.maximum(cnt, 1.0)
    h = (lax.dot(u_ref[...], W1_ref[0:D_U, :],
                 precision=lax.Precision.HIGHEST,
                 preferred_element_type=jnp.float32)
         + lax.dot(mean, W1_ref[D_U:, :],
                   precision=lax.Precision.HIGHEST,
                   preferred_element_type=jnp.float32)
         + b1_ref[...])
    h = jnp.where(h >= 0, h, 0.01 * h)
    mu = jnp.mean(h, axis=-1, keepdims=True)
    var = jnp.mean((h - mu) ** 2, axis=-1, keepdims=True)
    h = (h - mu) * lax.rsqrt(var + 1e-5) * g_ref[...] + be_ref[...]
    o_ref[...] = (lax.dot(h, W2_ref[...],
                          precision=lax.Precision.HIGHEST,
                          preferred_element_type=jnp.float32)
                  + b2_ref[...])


def kernel(x, edge_index, edge_attr, u, batch, W1, b1, gamma, beta, W2, b2):
    del edge_index, edge_attr
    seg = batch.astype(jnp.int32)
    seg_pad = jnp.concatenate(
        [seg, jnp.full((N_PAD - N_NODES,), NUM_GRAPHS, jnp.int32)])
    zacc = jnp.zeros((NUM_GRAPHS + 1, D_NODE), jnp.float32)

    counts = pl.pallas_call(
        _cnt_body,
        grid=(CNT_STEPS,),
        in_specs=[pl.BlockSpec((1, 1, CNT_BLK), lambda k: (k, 0, 0))],
        out_specs=pl.BlockSpec((CNT_HI, CNT_LO), lambda k: (0, 0)),
        out_shape=jax.ShapeDtypeStruct((CNT_HI, CNT_LO), jnp.float32),
    )(seg_pad.reshape(CNT_STEPS, 1, CNT_BLK))
    counts = counts.reshape(NUM_GRAPHS, 1)

    part_sums = _sc_segment_sum(x, seg_pad.reshape(NCHUNK, SUB, 128), zacc)

    out = pl.pallas_call(
        _mlp_body,
        out_shape=jax.ShapeDtypeStruct((NUM_GRAPHS, D_REP), jnp.float32),
    )(part_sums, counts, u, W1,
      b1.reshape(1, D_REP), gamma.reshape(1, D_REP),
      beta.reshape(1, D_REP), W2, b2.reshape(1, D_REP))
    return out


# speedup vs baseline: 8.5436x; 1.0584x over previous
import functools

import jax
import jax.numpy as jnp
import numpy as np
from jax import lax
from jax.experimental import pallas as pl
from jax.experimental.pallas import tpu as pltpu
from jax.experimental.pallas import tpu_sc as plsc

NUM_GRAPHS = 512
N_NODES = 100000
D_NODE = 128
D_U = 64
D_REP = 128

NC = 2
NS = 16
NW = NC * NS

ROWS_IT = 256
SUB = ROWS_IT // 128
CNT_BLK = 4096
CNT_STEPS = 25
N_PAD = CNT_BLK * CNT_STEPS
NCHUNK = N_PAD // ROWS_IT
FULL_IT = N_NODES // ROWS_IT
TAIL_BASE = FULL_IT * ROWS_IT
TAIL_ROWS = N_NODES - TAIL_BASE
BASE_CH = FULL_IT // NW
EXTRA = FULL_IT - BASE_CH * NW
MAX_CH = BASE_CH + (1 if EXTRA else 0)
OUTER = (MAX_CH + 1) // 2

CNT_HI = 32
CNT_LO = 16


TAIL_WID = 7


def _sc_body(x_hbm, seg_hbm, zacc_hbm, sums_hbm,
             idx_v, rows_v, acc_sh,
             sem_i0, sem_i1, sem_r0, sem_r1, sem_s0, sem_s1):
    c = lax.axis_index("c")
    s = lax.axis_index("s")
    wid = s * NC + c
    sem_i = (sem_i0, sem_i1)
    sem_r = (sem_r0, sem_r1)
    sem_s = (sem_s0, sem_s1)

    @pl.when(s == 0)
    def _():
        pltpu.sync_copy(zacc_hbm, acc_sh)

    plsc.subcore_barrier()

    n_ch = BASE_CH + jnp.where(wid < EXTRA, 1, 0)

    def start_in(ch, b):
        pltpu.async_copy(seg_hbm.at[ch], idx_v.at[b], sem_i[b])
        pltpu.async_copy(x_hbm.at[pl.ds(ch * ROWS_IT, ROWS_IT)],
                         rows_v.at[b], sem_r[b])

    def wait_in(ch, b):
        pltpu.make_async_copy(seg_hbm.at[ch], idx_v.at[b], sem_i[b]).wait()
        pltpu.make_async_copy(x_hbm.at[pl.ds(ch * ROWS_IT, ROWS_IT)],
                              rows_v.at[b], sem_r[b]).wait()

    def fire_scatter(b):
        for j in range(SUB):
            pltpu.async_copy(rows_v.at[b].at[pl.ds(j * 128, 128)],
                             acc_sh.at[idx_v.at[b, j]], sem_s[b], add=True)

    def wait_scatter(b):
        for j in range(SUB):
            pltpu.make_async_copy(rows_v.at[b].at[pl.ds(j * 128, 128)],
                                  acc_sh.at[idx_v.at[b, j]],
                                  sem_s[b]).wait()

    start_in(wid, 0)

    def outer(g, carry):
        for b in range(2):
            k = g * 2 + b
            ch = k * NW + wid

            @pl.when(k < n_ch)
            def _():
                wait_in(ch, b)
                fire_scatter(b)

                @pl.when(k + 1 < n_ch)
                def _():
                    @pl.when(k >= 1)
                    def _():
                        wait_scatter(1 - b)

                    start_in(ch + NW, 1 - b)
        return carry

    lax.fori_loop(0, OUTER, outer, 0)

    wait_scatter(0)
    wait_scatter(1)

    @pl.when(wid == TAIL_WID)
    def _():
        pltpu.sync_copy(seg_hbm.at[FULL_IT], idx_v.at[0])
        pltpu.sync_copy(x_hbm.at[pl.ds(TAIL_BASE, TAIL_ROWS)],
                        rows_v.at[0].at[pl.ds(0, TAIL_ROWS)])
        fire_scatter(0)
        wait_scatter(0)

    plsc.subcore_barrier()

    @pl.when(s == 0)
    def _():
        pltpu.sync_copy(acc_sh.at[pl.ds(0, NUM_GRAPHS)], sums_hbm.at[c])


_sc_segment_sum = functools.partial(
    pl.kernel,
    out_type=jax.ShapeDtypeStruct((NC, NUM_GRAPHS, D_NODE), jnp.float32),
    mesh=plsc.VectorSubcoreMesh(core_axis_name="c", subcore_axis_name="s",
                                num_cores=NC, num_subcores=NS),
    scratch_types=(
        pltpu.VMEM((2, SUB, 128), jnp.int32),
        pltpu.VMEM((2, ROWS_IT, D_NODE), jnp.float32),
        pltpu.VMEM_SHARED((NUM_GRAPHS + 1, D_NODE), jnp.float32),
        pltpu.SemaphoreType.DMA,
        pltpu.SemaphoreType.DMA,
        pltpu.SemaphoreType.DMA,
        pltpu.SemaphoreType.DMA,
        pltpu.SemaphoreType.DMA,
        pltpu.SemaphoreType.DMA,
    ),
)(_sc_body)


def _cnt_body(ids_ref, o_ref):
    k = pl.program_id(0)
    ids = ids_ref[0]
    hi = ids >> 4
    lo = ids & 15
    a = (hi == lax.broadcasted_iota(jnp.int32, (CNT_HI, 1), 0))
    b = (lo == lax.broadcasted_iota(jnp.int32, (CNT_LO, 1), 0))
    contrib = lax.dot_general(a.astype(jnp.float32), b.astype(jnp.float32),
                              (((1,), (1,)), ((), ())),
                              preferred_element_type=jnp.float32)

    @pl.when(k == 0)
    def _():
        o_ref[...] = jnp.zeros_like(o_ref)

    o_ref[...] += contrib


def _mlp_body(ps_ref, pc_ref, u_ref, W1_ref, b1_ref, g_ref, be_ref,
              W2_ref, b2_ref, o_ref):
    sums = ps_ref[0] + ps_ref[1]
    cnt = pc_ref[...]
    mean = sums / jnp.maximum(cnt, 1.0)
    h = (lax.dot(u_ref[...], W1_ref[0:D_U, :],
                 precision=lax.Precision.HIGHEST,
                 preferred_element_type=jnp.float32)
         + lax.dot(mean, W1_ref[D_U:, :],
                   precision=lax.Precision.HIGHEST,
                   preferred_element_type=jnp.float32)
         + b1_ref[...])
    h = jnp.where(h >= 0, h, 0.01 * h)
    mu = jnp.mean(h, axis=-1, keepdims=True)
    var = jnp.mean((h - mu) ** 2, axis=-1, keepdims=True)
    h = (h - mu) * lax.rsqrt(var + 1e-5) * g_ref[...] + be_ref[...]
    o_ref[...] = (lax.dot(h, W2_ref[...],
                          precision=lax.Precision.HIGHEST,
                          preferred_element_type=jnp.float32)
                  + b2_ref[...])


def kernel(x, edge_index, edge_attr, u, batch, W1, b1, gamma, beta, W2, b2):
    del edge_index, edge_attr
    seg = batch.astype(jnp.int32)
    seg_pad = jnp.concatenate(
        [seg, jnp.full((N_PAD - N_NODES,), NUM_GRAPHS, jnp.int32)])
    zacc = np.zeros((NUM_GRAPHS + 1, D_NODE), np.float32)

    counts = pl.pallas_call(
        _cnt_body,
        grid=(CNT_STEPS,),
        in_specs=[pl.BlockSpec((1, 1, CNT_BLK), lambda k: (k, 0, 0))],
        out_specs=pl.BlockSpec((CNT_HI, CNT_LO), lambda k: (0, 0)),
        out_shape=jax.ShapeDtypeStruct((CNT_HI, CNT_LO), jnp.float32),
    )(seg_pad.reshape(CNT_STEPS, 1, CNT_BLK))
    counts = counts.reshape(NUM_GRAPHS, 1)

    part_sums = _sc_segment_sum(x, seg_pad.reshape(NCHUNK, SUB, 128), zacc)

    out = pl.pallas_call(
        _mlp_body,
        out_shape=jax.ShapeDtypeStruct((NUM_GRAPHS, D_REP), jnp.float32),
    )(part_sums, counts, u, W1,
      b1.reshape(1, D_REP), gamma.reshape(1, D_REP),
      beta.reshape(1, D_REP), W2, b2.reshape(1, D_REP))
    return out
